# Initial kernel scaffold; baseline (speedup 1.0000x reference)
import jax
import jax.numpy as jnp
from jax.experimental import pallas as pl


def kernel(time_idx, day_idx, time_table, day_table):
    raise NotImplementedError("write your pallas kernel here")


# speedup vs baseline: 8.8284x; 8.8284x over previous
import functools

import jax
import jax.numpy as jnp
from jax import lax
from jax.experimental import pallas as pl
from jax.experimental.pallas import tpu as pltpu
from jax.experimental.pallas import tpu_sc as plsc

_NC = 2
_NS = 16
_NW = _NC * _NS

_T_ROWS = 288
_D_ROWS = 7
_F_ROWS = _T_ROWS * _D_ROWS
_EMB = 32
_OUT_W = 2 * _EMB


def _fused_table_body(t64_ref, d64_ref, o_ref):
    r = lax.broadcasted_iota(jnp.int32, (_F_ROWS, 1), 0)
    t_of_r = r // _D_ROWS
    d_of_r = r - t_of_r * _D_ROWS
    col_t = lax.broadcasted_iota(jnp.int32, (_F_ROWS, _T_ROWS), 1)
    oh_t = (col_t == t_of_r).astype(jnp.float32)
    col_d = lax.broadcasted_iota(jnp.int32, (_F_ROWS, 8), 1)
    oh_d = (col_d == d_of_r).astype(jnp.float32)
    o_ref[...] = (
        jnp.dot(oh_t, t64_ref[...], preferred_element_type=jnp.float32)
        + jnp.dot(oh_d, d64_ref[...], preferred_element_type=jnp.float32)
    )


def _build_fused_table(time_table, day_table):
    t64 = jnp.pad(time_table.astype(jnp.float32), ((0, 0), (0, _EMB)))
    d64 = jnp.pad(day_table.astype(jnp.float32), ((0, 1), (_EMB, 0)))
    return pl.pallas_call(
        _fused_table_body,
        out_shape=jax.ShapeDtypeStruct((_F_ROWS, _OUT_W), jnp.float32),
    )(t64, d64)


def _make_sc_lookup(n_rows, chunk_rows):
    assert chunk_rows % 1024 == 0
    r_per_chunk = chunk_rows // 128
    rows_w = n_rows // _NW
    assert rows_w % chunk_rows == 0
    n_chunks = rows_w // chunk_rows
    irows_w = rows_w // 128

    mesh = plsc.VectorSubcoreMesh(
        core_axis_name="c", subcore_axis_name="s",
        num_cores=_NC, num_subcores=_NS,
    )

    @functools.partial(
        pl.kernel,
        out_type=jax.ShapeDtypeStruct((n_rows, _OUT_W), jnp.float32),
        mesh=mesh,
        compiler_params=pltpu.CompilerParams(use_tc_tiling_on_sc=False),
        scratch_types=[
            pltpu.VMEM((r_per_chunk, 128), jnp.int32),
            pltpu.VMEM((r_per_chunk, 128), jnp.int32),
            pltpu.VMEM((r_per_chunk, 128), jnp.int32),
            pltpu.VMEM((chunk_rows, _OUT_W), jnp.float32),
            pltpu.SemaphoreType.DMA,
        ],
    )
    def sc_lookup(t_hbm, d_hbm, fused_hbm, out_hbm, it_v, id_v, ic_v, rows_v, sem):
        wid = lax.axis_index("s") * _NC + lax.axis_index("c")
        irow0_w = wid * irows_w

        def chunk(i, carry):
            irow0 = irow0_w + i * r_per_chunk
            pltpu.sync_copy(t_hbm.at[pl.ds(irow0, r_per_chunk)], it_v)
            pltpu.sync_copy(d_hbm.at[pl.ds(irow0, r_per_chunk)], id_v)

            def combine_row(r, carry2):
                for l in range(8):
                    sl = pl.ds(l * 16, 16)
                    ic_v[r, sl] = it_v[r, sl] * _D_ROWS + id_v[r, sl]
                return carry2

            lax.fori_loop(0, r_per_chunk, combine_row, 0)

            copies = [
                pltpu.async_copy(
                    fused_hbm.at[ic_v.at[r]],
                    rows_v.at[pl.ds(r * 128, 128)],
                    sem,
                )
                for r in range(r_per_chunk)
            ]
            for cp in copies:
                cp.wait()
            pltpu.sync_copy(rows_v, out_hbm.at[pl.ds(irow0 * 128, chunk_rows)])
            return carry

        lax.fori_loop(0, n_chunks, chunk, 0)

    return sc_lookup


def kernel(time_idx, day_idx, time_table, day_table):
    b, s = time_idx.shape
    n_rows = b * s
    t2 = time_idx.reshape(n_rows // 128, 128).astype(jnp.int32)
    d2 = day_idx.reshape(n_rows // 128, 128).astype(jnp.int32)
    fused = _build_fused_table(time_table, day_table)
    out = _make_sc_lookup(n_rows, chunk_rows=1024)(t2, d2, fused)
    return out.reshape(b, s, _OUT_W)
